# baseline (speedup 1.0000x reference)
import jax
import jax.numpy as jnp
from jax import lax
from jax.experimental import pallas as pl
from jax.experimental.pallas import tpu as pltpu, tpu_sc as plsc

VOC_SIZE = 1000000
EMBED_DIM = 64
BATCH = 16384

NUM_CORES = 2
NUM_SUBCORES = 16
NUM_WORKERS = NUM_CORES * NUM_SUBCORES
B_PER_W = BATCH // NUM_WORKERS
B_PER_SC = BATCH // NUM_CORES
NVT = (VOC_SIZE + 127) // 128
NVEC = B_PER_SC // 16
RANGE_CAP = 489
WCAP = 512
LIST_CAP = 672
NSUP_MAX = WCAP // 16
SENT_ROW = B_PER_SC
FLAG_N = 512


def _scalar(x):
    return x if x.ndim == 0 else x[0]


def _append2(l1_ref, l2_ref, off, v1, v2, m):
    pos = off + plsc.cumsum(jnp.where(m, 1, 0)) - 1
    plsc.store_scatter(l1_ref, [pos], v1, mask=m)
    plsc.store_scatter(l2_ref, [pos], v2, mask=m)
    return off + _scalar(plsc.all_reduce_population_count(m))


def _body(center_hbm, context_hbm, embT_hbm, ctxT_hbm, out_hbm,
          mvu, mbu, mvx, mbx, wpu, wpx,
          flg_u, flg_x, map_u, map_x, wl_u, wl_x,
          bv, bb, bw, asm_v, stage_u, stage_x, dump_v, out_v,
          cidx_v, xidx_v, ring, su_v, sv_v, smw, sem, sem2):
    cid = lax.axis_index("c")
    sid = lax.axis_index("s")
    lane = lax.iota(jnp.int32, 16)

    lo = sid * 488 + jnp.minimum(sid, 5)
    nrange = 488 + jnp.where(sid < 5, 1, 0)

    def phase_a(half):
        for h in range(NUM_SUBCORES // 2):
            hs = pl.ds(h * (NVEC // NUM_SUBCORES), NVEC // NUM_SUBCORES)
            hsrc = half * (NUM_SUBCORES // 2) + h
            pltpu.sync_copy(center_hbm.at[cid, hsrc], cidx_v.at[hs])
            pltpu.sync_copy(context_hbm.at[cid, hsrc], xidx_v.at[hs])

        def filt(i, offs):
            off_u, off_x = offs
            b16 = (half * (NVEC // 2) + i) * 16 + lane
            v16 = cidx_v[i, pl.ds(0, 16)]
            vt = lax.shift_right_logical(v16, 7)
            mu = (vt >= lo) & (vt < lo + nrange)
            plsc.store_scatter(
                flg_u, [jnp.clip(vt - lo, 0, FLAG_N - 1)],
                jnp.full((16,), 1, jnp.int32), mask=mu)
            off_u = _append2(mvu, mbu, off_u, v16, b16, mu)
            x16 = xidx_v[i, pl.ds(0, 16)]
            xt = lax.shift_right_logical(x16, 7)
            mx = (xt >= lo) & (xt < lo + nrange)
            plsc.store_scatter(
                flg_x, [jnp.clip(xt - lo, 0, FLAG_N - 1)],
                jnp.full((16,), 1, jnp.int32), mask=mx)
            off_x = _append2(mvx, mbx, off_x, x16, b16, mx)
            return (off_u, off_x)

        z16 = jnp.zeros((16,), jnp.int32)
        for j in range(FLAG_N // 16):
            flg_u[pl.ds(j * 16, 16)] = z16
            flg_x[pl.ds(j * 16, 16)] = z16
        return lax.fori_loop(0, NVEC // 2,
                             filt, (jnp.int32(0), jnp.int32(0)))

    def compact(flg_ref, map_ref, wl_ref):
        def body(j, cnt):
            f16 = flg_ref[pl.ds(j * 16, 16)]
            m = f16 == 1
            pos = cnt + plsc.cumsum(jnp.where(m, 1, 0)) - 1
            plsc.store_scatter(wl_ref, [pos], j * 16 + lane, mask=m)
            map_ref[pl.ds(j * 16, 16)] = pos
            return cnt + _scalar(plsc.all_reduce_population_count(m))
        return lax.fori_loop(0, FLAG_N // 16, body, jnp.int32(0))


    def attach(lv_ref, wp_ref, map_ref, nn):
        def body(j, c):
            v16 = lv_ref[pl.ds(j * 16, 16)]
            lvt = jnp.clip(
                lax.shift_right_logical(v16, 7) - lo, 0, FLAG_N - 1)
            wp_ref[pl.ds(j * 16, 16)] = plsc.load_gather(map_ref, [lvt])
            return c
        lax.fori_loop(0, (nn + 15) >> 4, body, 0)


    def phase_d(half, nu, nx, mw_u, mw_x):
        hlo = half * (B_PER_SC // 2)
        hhi = hlo + B_PER_SC // 2

        def table_loop(tab_hbm, wl_ref, lv_ref, lb_ref, wp_ref, nn, mw,
                       stage_ref):
            def fire(bi):
                par = bi % 2
                vt = smw[bi]
                base = pl.multiple_of(lax.shift_left(vt, 7), 128)
                pltpu.async_copy(
                    tab_hbm.at[:, :, pl.ds(base, 128)],
                    ring.at[par, 0], sem)

            nsup = (mw + 15) >> 4

            def sup(s, carry):
                @pl.when(s < nsup)
                def _():
                    wvec0 = wl_ref[pl.ds(s * 16, 16)]
                    wvec_t = jnp.clip(wvec0 + lo, 0, NVT - 1)
                    sn = jnp.minimum(s + 1, NSUP_MAX - 1)
                    wvecn0 = wl_ref[pl.ds(sn * 16, 16)]
                    wvecn_t = jnp.clip(wvecn0 + lo, 0, NVT - 1)
                    for k in range(16):
                        smw[k] = wvec_t[k]
                        smw[16 + k] = wvecn_t[k]

                    @pl.when(s == 0)
                    def _():
                        fire(0)
                        fire(1)

                    def subfn(sub, c2):
                        par = lax.rem(sub, 2)
                        q = s * 16 + sub
                        pltpu.make_async_copy(
                            tab_hbm.at[:, :, pl.ds(0, 128)],
                            ring.at[0, 0], sem).wait()
                        @pl.when((s > 0) | (sub > 1))
                        def _():
                            pltpu.make_async_copy(
                                asm_v, stage_ref.at[pl.ds(0, 2048)],
                                sem2).wait()
                        sentv = smw[sub] * 128
                        for t in range(2):
                            bv[pl.ds(t * 16, 16)] = jnp.full(
                                (16,), sentv, jnp.int32)
                            bb[pl.ds(t * 16, 16)] = jnp.full(
                                (16,), SENT_ROW, jnp.int32)
                            bw[pl.ds(t * 16, 16)] = jnp.full(
                                (16,), q, jnp.int32)

                        def sel(j, offb):
                            v16 = lv_ref[pl.ds(j * 16, 16)]
                            b16 = lb_ref[pl.ds(j * 16, 16)]
                            w16 = wp_ref[pl.ds(j * 16, 16)]
                            m = ((w16 == q)
                                 & (j * 16 + lane < nn))
                            pos = offb + plsc.cumsum(
                                jnp.where(m, 1, 0)) - 1
                            plsc.store_scatter(bv, [pos], v16, mask=m)
                            plsc.store_scatter(bb, [pos], b16, mask=m)
                            plsc.store_scatter(bw, [pos], w16, mask=m)
                            return offb + _scalar(
                                plsc.all_reduce_population_count(m))

                        lax.fori_loop(0, (nn + 15) >> 4, sel, jnp.int32(0))

                        p_spl = jnp.full((16,), par, jnp.int32)
                        slot = jnp.zeros((16,), jnp.int32)
                        for t in range(2):
                            v16 = bv[pl.ds(t * 16, 16)]
                            b16 = bb[pl.ds(t * 16, 16)]
                            li = v16 & 127
                            rowbase = (t * 16 + lane) * EMBED_DIM

                            def cgrp(cg, cc2):
                                g_spl = jnp.full((16,), cg, jnp.int32)
                                for cc in range(8):
                                    s_spl = jnp.full((16,), cc, jnp.int32)
                                    wv = plsc.load_gather(
                                        ring, [p_spl, slot, g_spl, s_spl, li])
                                    plsc.store_scatter(
                                        asm_v, [rowbase + cg * 8 + cc], wv)
                                return cc2

                            lax.fori_loop(0, 8, cgrp, 0)
                            for k in range(16):
                                bk = b16[k]
                                src = asm_v.at[pl.ds(
                                    (t * 16 + k) * EMBED_DIM, EMBED_DIM)]

                                inh = (bk >= hlo) & (bk < hhi)

                                @pl.when(inh)
                                def _():
                                    dst = pl.multiple_of(
                                        (bk - hlo) * EMBED_DIM, 64)
                                    pltpu.async_copy(
                                        src,
                                        stage_ref.at[pl.ds(dst, EMBED_DIM)],
                                        sem2)

                                @pl.when(jnp.logical_not(inh))
                                def _():
                                    pltpu.async_copy(
                                        src, dump_v, sem2)

                        fire(sub + 2)
                        return c2

                    lax.fori_loop(0, 16, subfn, 0)

                return carry

            lax.fori_loop(0, NSUP_MAX, sup, 0)

            @pl.when(nsup > 0)
            def _():
                for _ in range(2):
                    pltpu.make_async_copy(
                        tab_hbm.at[:, :, pl.ds(0, 128)],
                        ring.at[0, 0], sem).wait()
                for _ in range(2):
                    pltpu.make_async_copy(
                        asm_v, stage_ref.at[pl.ds(0, 2048)], sem2).wait()

        table_loop(embT_hbm, wl_u, mvu, mbu, wpu, nu, mw_u, stage_u)
        table_loop(ctxT_hbm, wl_x, mvx, mbx, wpx, nx, mw_x, stage_x)

    CH = 32
    E_ROWS = B_PER_SC // 2 // NUM_SUBCORES

    def phase_e(half):
        masks = [lane == i for i in range(16)]

        def chunk(ch, carry):
            base = (sid * E_ROWS + ch * CH) * EMBED_DIM
            pltpu.sync_copy(stage_u.at[pl.ds(base, CH * EMBED_DIM)], su_v)
            pltpu.sync_copy(stage_x.at[pl.ds(base, CH * EMBED_DIM)], sv_v)

            def group(g, c2):
                res = jnp.zeros((16,), jnp.float32)
                for i in range(16):
                    r = (g * 16 + i) * EMBED_DIM
                    w = su_v[pl.ds(r, 16)] * sv_v[pl.ds(r, 16)]
                    for cc in range(1, EMBED_DIM // 16):
                        w = w + (su_v[pl.ds(r + cc * 16, 16)]
                                 * sv_v[pl.ds(r + cc * 16, 16)])
                    sc = jnp.sum(w)
                    res = jnp.where(masks[i],
                                    jnp.full((16,), sc, jnp.float32), res)
                out_v[pl.ds(ch * CH + g * 16, 16)] = res
                return c2

            lax.fori_loop(0, CH // 16, group, 0)
            return carry

        lax.fori_loop(0, E_ROWS // CH, chunk, 0)
        dst = cid * B_PER_SC + half * (B_PER_SC // 2) + sid * E_ROWS
        pltpu.sync_copy(out_v.at[pl.ds(0, E_ROWS)],
                        out_hbm.at[pl.ds(dst, E_ROWS)])

    def half_pass(half, carry):
        nu, nx = phase_a(half)
        mw_u = compact(flg_u, map_u, wl_u)
        mw_x = compact(flg_x, map_x, wl_x)
        attach(mvu, wpu, map_u, nu)
        attach(mvx, wpx, map_x, nx)
        phase_d(half, nu, nx, mw_u, mw_x)
        plsc.subcore_barrier()
        phase_e(half)
        plsc.subcore_barrier()
        return carry

    lax.fori_loop(0, 2, half_pass, 0)


@jax.jit
def kernel(center, context, emb_weight, ctx_weight):
    mesh = plsc.VectorSubcoreMesh(core_axis_name="c", subcore_axis_name="s")
    run = pl.kernel(
        _body,
        out_type=jax.ShapeDtypeStruct((BATCH,), jnp.float32),
        mesh=mesh,
        compiler_params=pltpu.CompilerParams(
            use_tc_tiling_on_sc=True, needs_layout_passes=False),
        scratch_types=[
            pltpu.VMEM((LIST_CAP,), jnp.int32),
            pltpu.VMEM((LIST_CAP,), jnp.int32),
            pltpu.VMEM((LIST_CAP,), jnp.int32),
            pltpu.VMEM((LIST_CAP,), jnp.int32),
            pltpu.VMEM((LIST_CAP,), jnp.int32),
            pltpu.VMEM((LIST_CAP,), jnp.int32),
            pltpu.VMEM((FLAG_N,), jnp.int32),
            pltpu.VMEM((FLAG_N,), jnp.int32),
            pltpu.VMEM((FLAG_N,), jnp.int32),
            pltpu.VMEM((FLAG_N,), jnp.int32),
            pltpu.VMEM((WCAP,), jnp.int32),
            pltpu.VMEM((WCAP,), jnp.int32),
            pltpu.VMEM((32,), jnp.int32),
            pltpu.VMEM((32,), jnp.int32),
            pltpu.VMEM((32,), jnp.int32),
            pltpu.VMEM((32 * EMBED_DIM,), jnp.float32),
            pltpu.VMEM_SHARED((B_PER_SC // 2 * EMBED_DIM,), jnp.float32),
            pltpu.VMEM_SHARED((B_PER_SC // 2 * EMBED_DIM,), jnp.float32),
            pltpu.VMEM_SHARED((EMBED_DIM,), jnp.float32),
            pltpu.VMEM((B_PER_W,), jnp.float32),
            pltpu.VMEM((NVEC // 2, 16), jnp.int32),
            pltpu.VMEM((NVEC // 2, 16), jnp.int32),
            pltpu.VMEM((2, 1, 8, 8, 128), jnp.float32),
            pltpu.VMEM((32 * EMBED_DIM,), jnp.float32),
            pltpu.VMEM((32 * EMBED_DIM,), jnp.float32),
            pltpu.SMEM((32,), jnp.int32),
            pltpu.SemaphoreType.DMA,
            pltpu.SemaphoreType.DMA,
        ],
    )
    center_c = center.astype(jnp.int32).reshape(
        NUM_CORES, NUM_SUBCORES, NVEC // NUM_SUBCORES, 16)
    context_c = context.astype(jnp.int32).reshape(
        NUM_CORES, NUM_SUBCORES, NVEC // NUM_SUBCORES, 16)
    embT3 = emb_weight.T.reshape(8, 8, VOC_SIZE)
    ctxT3 = ctx_weight.T.reshape(8, 8, VOC_SIZE)
    return run(center_c, context_c, embT3, ctxT3)


# speedup vs baseline: 4.1168x; 4.1168x over previous
import jax
import jax.numpy as jnp
from jax import lax
from jax.experimental import pallas as pl
from jax.experimental.pallas import tpu as pltpu, tpu_sc as plsc

VOC_SIZE = 1000000
EMBED_DIM = 64
BATCH = 16384

NUM_CORES = 2
NUM_SUBCORES = 16
NUM_WORKERS = NUM_CORES * NUM_SUBCORES
B_PER_W = BATCH // NUM_WORKERS
SUPER = B_PER_W // 16
SUBS = 8


def _fire(embT_hbm, ctxT_hbm, u_bufs, v_bufs, sem, ivec_c, ivec_x, sub):
    par = sub & 1
    for k in range(2):
        cu = ivec_c[sub * 2 + k]
        cx = ivec_x[sub * 2 + k]
        bu = pl.multiple_of(lax.shift_left(lax.shift_right_logical(cu, 7), 7), 128)
        bx = pl.multiple_of(lax.shift_left(lax.shift_right_logical(cx, 7), 7), 128)
        pltpu.async_copy(embT_hbm.at[:, :, pl.ds(bu, 128)], u_bufs.at[par, k], sem)
        pltpu.async_copy(ctxT_hbm.at[:, :, pl.ds(bx, 128)], v_bufs.at[par, k], sem)


def _drain(embT_hbm, u_bufs, sem):
    for _ in range(4):
        pltpu.make_async_copy(
            embT_hbm.at[:, :, pl.ds(0, 128)], u_bufs.at[0, 0], sem).wait()


def _sc_body(center_hbm, context_hbm, embT_hbm, ctxT_hbm, out_hbm,
             cidx_v, xidx_v, u_bufs, v_bufs, out_v, sem):
    wid = lax.axis_index("s") * NUM_CORES + lax.axis_index("c")

    pltpu.sync_copy(center_hbm.at[wid], cidx_v)
    pltpu.sync_copy(context_hbm.at[wid], xidx_v)

    lane = lax.iota(jnp.int32, 16)
    s_vec = lane & 7
    g_vecs = [(lane >> 3) + 2 * cg for cg in range(4)]
    masks = [lane == i for i in range(16)]

    ivec_c0 = cidx_v[0, pl.ds(0, 16)]
    ivec_x0 = xidx_v[0, pl.ds(0, 16)]
    _fire(embT_hbm, ctxT_hbm, u_bufs, v_bufs, sem, ivec_c0, ivec_x0, 0)
    _fire(embT_hbm, ctxT_hbm, u_bufs, v_bufs, sem, ivec_c0, ivec_x0, 1)

    def super_group(s, carry):
        ivec_c = cidx_v[s, pl.ds(0, 16)]
        ivec_x = xidx_v[s, pl.ds(0, 16)]
        sn = jnp.minimum(s + 1, SUPER - 1)
        ivec_cn = cidx_v[sn, pl.ds(0, 16)]
        ivec_xn = xidx_v[sn, pl.ds(0, 16)]
        res = jnp.zeros((16,), jnp.float32)
        for sub in range(SUBS):
            par = sub & 1
            _drain(embT_hbm, u_bufs, sem)
            for k in range(2):
                lu = jnp.full((16,), ivec_c[sub * 2 + k] & 127, jnp.int32)
                lx = jnp.full((16,), ivec_x[sub * 2 + k] & 127, jnp.int32)
                pv = jnp.full((16,), par, jnp.int32)
                kv = jnp.full((16,), k, jnp.int32)
                w = jnp.zeros((16,), jnp.float32)
                for cg in range(4):
                    u16 = plsc.load_gather(u_bufs, [pv, kv, g_vecs[cg], s_vec, lu])
                    v16 = plsc.load_gather(v_bufs, [pv, kv, g_vecs[cg], s_vec, lx])
                    w = w + u16 * v16
                sc = jnp.sum(w)
                res = jnp.where(masks[sub * 2 + k],
                                jnp.full((16,), sc, jnp.float32), res)
            if sub + 2 < SUBS:
                _fire(embT_hbm, ctxT_hbm, u_bufs, v_bufs, sem,
                      ivec_c, ivec_x, sub + 2)
            else:

                @pl.when(s < SUPER - 1)
                def _():
                    _fire(embT_hbm, ctxT_hbm, u_bufs, v_bufs, sem,
                          ivec_cn, ivec_xn, sub + 2 - SUBS)

        out_v[pl.ds(s * 16, 16)] = res
        return carry

    lax.fori_loop(0, SUPER, super_group, 0)

    pltpu.sync_copy(out_v, out_hbm.at[pl.ds(wid * B_PER_W, B_PER_W)])


@jax.jit
def kernel(center, context, emb_weight, ctx_weight):
    mesh = plsc.VectorSubcoreMesh(core_axis_name="c", subcore_axis_name="s")
    run = pl.kernel(
        _sc_body,
        out_type=jax.ShapeDtypeStruct((BATCH,), jnp.float32),
        mesh=mesh,
        compiler_params=pltpu.CompilerParams(
            use_tc_tiling_on_sc=True, needs_layout_passes=False),
        scratch_types=[
            pltpu.VMEM((SUPER, 16), jnp.int32),
            pltpu.VMEM((SUPER, 16), jnp.int32),
            pltpu.VMEM((2, 2, 8, 8, 128), jnp.float32),
            pltpu.VMEM((2, 2, 8, 8, 128), jnp.float32),
            pltpu.VMEM((B_PER_W,), jnp.float32),
            pltpu.SemaphoreType.DMA,
        ],
    )
    center_c = center.astype(jnp.int32).reshape(NUM_WORKERS, SUPER, 16)
    context_c = context.astype(jnp.int32).reshape(NUM_WORKERS, SUPER, 16)
    embT3 = emb_weight.T.reshape(8, 8, VOC_SIZE)
    ctxT3 = ctx_weight.T.reshape(8, 8, VOC_SIZE)
    return run(center_c, context_c, embT3, ctxT3)
